# Initial kernel scaffold; baseline (speedup 1.0000x reference)
import jax
import jax.numpy as jnp
from jax.experimental import pallas as pl


def kernel(coords, features, len_batch):
    raise NotImplementedError("write your pallas kernel here")


# speedup vs baseline: 107.5656x; 107.5656x over previous
import jax
import jax.numpy as jnp
from jax.experimental import pallas as pl


def _score_kernel(x_ref, o_ref):
    b = x_ref.shape[0]
    for i in range(b):
        f = jnp.maximum(x_ref[i], 0.0)
        m = jnp.max(f, axis=1, keepdims=True)
        beta = f / m
        e = jnp.exp(f)
        alpha = e / e[0:1, :]
        g = jnp.max(alpha * beta, axis=1)
        o_ref[i] = g / jnp.sqrt(jnp.sum(g * g))


def kernel(coords, features, len_batch):
    b, n, _ = features.shape
    out = pl.pallas_call(
        _score_kernel,
        out_shape=jax.ShapeDtypeStruct((b, n), features.dtype),
    )(features)
    out = out.reshape(b * n)
    return out + 0.0 * jnp.asarray(len_batch, dtype=out.dtype)
